# baseline (speedup 1.0000x reference)
import numpy as np
import jax
import jax.numpy as jnp
from jax import lax
from jax.experimental import pallas as pl
from jax.experimental.pallas import tpu as pltpu
from jax.experimental.pallas import tpu_sc as plsc

N_RES = 100000
WORDS = N_RES * 12
GROUPS = WORDS // 48

_info = plsc.get_sparse_core_info()
NC = _info.num_cores
NS = _info.num_subcores
NW = NC * NS

G_MAIN = -(-GROUPS // NW)
G_LAST = GROUPS - G_MAIN * (NW - 1)
W_MAIN = G_MAIN * 48
W_LAST = G_LAST * 48
PAD = 8


def _body(x_hbm, out_hbm, buf, obuf):
    wid = lax.axis_index("s") * NC + lax.axis_index("c")
    base = wid * W_MAIN

    @pl.when(wid < NW - 1)
    def _():
        pltpu.sync_copy(x_hbm.at[pl.ds(base, W_MAIN)],
                        buf.at[pl.ds(PAD, W_MAIN)])

    @pl.when(wid == NW - 1)
    def _():
        pltpu.sync_copy(x_hbm.at[pl.ds(base, W_LAST)],
                        buf.at[pl.ds(PAD, W_LAST)])

    lanes = lax.iota(jnp.int32, 16)
    masks = []
    for j in range(3):
        col = jnp.remainder(lanes + (16 * j) % 12, 12)
        take_fwd = (col >= 3) & (col <= 5)
        take_bwd = (col >= 6) & (col <= 8)
        masks.append((take_fwd, take_bwd))

    ngroups = jnp.where(wid == NW - 1, G_LAST, G_MAIN)

    def step(k, carry):
        s = PAD + k * 48
        for j in range(3):
            off = s + 16 * j
            ident = buf[pl.ds(off, 16)]
            fwd = buf[pl.ds(off + 3, 16)]
            bwd = buf[pl.ds(off - 3, 16)]
            mf, mb = masks[j]
            out = jnp.where(mf, fwd, jnp.where(mb, bwd, ident))
            obuf[pl.ds(k * 48 + 16 * j, 16)] = out
        return carry

    lax.fori_loop(0, ngroups, step, 0)

    @pl.when(wid < NW - 1)
    def _():
        pltpu.sync_copy(obuf.at[pl.ds(0, W_MAIN)],
                        out_hbm.at[pl.ds(base, W_MAIN)])

    @pl.when(wid == NW - 1)
    def _():
        pltpu.sync_copy(obuf.at[pl.ds(0, W_LAST)],
                        out_hbm.at[pl.ds(base, W_LAST)])


def kernel(X):
    mesh = plsc.VectorSubcoreMesh(core_axis_name="c", subcore_axis_name="s")
    f = pl.kernel(
        _body,
        mesh=mesh,
        out_type=jax.ShapeDtypeStruct((WORDS,), jnp.float32),
        scratch_types=[
            pltpu.VMEM((PAD + W_MAIN + PAD,), jnp.float32),
            pltpu.VMEM((W_MAIN,), jnp.float32),
        ],
    )
    return f(X.reshape(-1)).reshape(N_RES, 4, 3)


# speedup vs baseline: 1.0180x; 1.0180x over previous
import jax
import jax.numpy as jnp
from jax import lax
from jax.experimental import pallas as pl
from jax.experimental.pallas import tpu as pltpu
from jax.experimental.pallas import tpu_sc as plsc

N_RES = 100000
WORDS = N_RES * 12
GROUPS = WORDS // 48

_info = plsc.get_sparse_core_info()
NC = _info.num_cores
NS = _info.num_subcores
NW = NC * NS

G_MAIN = -(-GROUPS // NW)
G_LAST = GROUPS - G_MAIN * (NW - 1)
W_MAIN = G_MAIN * 48
W_LAST = G_LAST * 48


def _body(x_hbm, out_hbm, buf, obuf):
    wid = lax.axis_index("s") * NC + lax.axis_index("c")
    base = wid * W_MAIN

    @pl.when(wid < NW - 1)
    def _():
        pltpu.sync_copy(x_hbm.at[pl.ds(base, W_MAIN)], buf)

    @pl.when(wid == NW - 1)
    def _():
        pltpu.sync_copy(x_hbm.at[pl.ds(base, W_LAST)],
                        buf.at[pl.ds(0, W_LAST)])

    perms = []
    for j in range(3):
        lanes = lax.iota(jnp.int32, 16) + 16 * j
        col = jnp.remainder(lanes, 12)
        shift = jnp.where((col >= 3) & (col <= 5), 3,
                          jnp.where((col >= 6) & (col <= 8), -3, 0))
        perms.append(lanes + shift)

    @plsc.parallel_loop(0, G_MAIN, unroll=8)
    def _(k):
        s = k * 48
        for j in range(3):
            v = plsc.load_gather(buf, [perms[j] + s])
            obuf[pl.ds(s + 16 * j, 16)] = v

    @pl.when(wid < NW - 1)
    def _():
        pltpu.sync_copy(obuf, out_hbm.at[pl.ds(base, W_MAIN)])

    @pl.when(wid == NW - 1)
    def _():
        pltpu.sync_copy(obuf.at[pl.ds(0, W_LAST)],
                        out_hbm.at[pl.ds(base, W_LAST)])


def kernel(X):
    mesh = plsc.VectorSubcoreMesh(core_axis_name="c", subcore_axis_name="s")
    f = pl.kernel(
        _body,
        mesh=mesh,
        compiler_params=pltpu.CompilerParams(needs_layout_passes=False),
        out_type=jax.ShapeDtypeStruct((WORDS,), jnp.float32),
        scratch_types=[
            pltpu.VMEM((W_MAIN,), jnp.float32),
            pltpu.VMEM((W_MAIN,), jnp.float32),
        ],
    )
    return f(X.reshape(-1)).reshape(N_RES, 4, 3)
